# baseline (speedup 1.0000x reference)
import jax
import jax.numpy as jnp
from jax import lax
from jax.experimental import pallas as pl
from jax.experimental.pallas import tpu as pltpu
from jax.experimental.pallas import tpu_sc as plsc

_N = 10000
_E = 320000
_D = 128

_NC = 2
_NS = 16
_NW = _NC * _NS
_K = 112
_EPT = -(-_E // (_NW * _K)) * _K
_E_PAD = _EPT * _NW
_CHUNKS = _EPT // _K
_C0 = 104
_C1 = 76
_CMX = max(_C0, _C1)
_NP = 10112
_ZR = _NP // _NS
_BM = 1000
_QH = 80
_BE = 4000


def _make_sc_agg():
    mesh = plsc.VectorSubcoreMesh(core_axis_name="c", subcore_axis_name="s")
    out_type = jax.ShapeDtypeStruct((_NC, _NP, _D), jnp.float32)
    scratch = [
        pltpu.VMEM((1, _K), jnp.int32),
        pltpu.VMEM((1, _K), jnp.int32),
        pltpu.VMEM((_CMX, _K), jnp.int32),
        pltpu.VMEM((_K, _D), jnp.float32),
        pltpu.VMEM((_K, _D), jnp.float32),
        pltpu.VMEM_SHARED((_NP, _D), jnp.float32),
        pltpu.SemaphoreType.DMA,
        pltpu.SemaphoreType.DMA,
        pltpu.SemaphoreType.DMA,
    ]

    def body(x_hbm, src_hbm, dst_hbm, out_p, sb0, sb1, didx, rows0, rows1,
             acc, gsem, ssem, isem):
        cid = lax.axis_index("c")
        sid = lax.axis_index("s")
        wid = sid * _NC + cid

        z16 = jnp.zeros((16,), jnp.float32)

        def zrow(i, carry):
            for j in range(_D // 16):
                rows0[i, pl.ds(j * 16, 16)] = z16
            return carry

        lax.fori_loop(0, _K, zrow, 0)
        for b in range(_ZR // _K):
            pltpu.sync_copy(rows0, acc.at[pl.ds(sid * _ZR + b * _K, _K)])
        rem = _ZR % _K
        pltpu.sync_copy(rows0.at[pl.ds(0, rem)],
                        acc.at[pl.ds(sid * _ZR + (_ZR // _K) * _K, rem)])
        wid2 = cid * _NS + sid
        nck = jnp.where(cid == 0, _C0, _C1)
        pltpu.sync_copy(dst_hbm.at[wid2], didx)
        pltpu.sync_copy(src_hbm.at[wid2, pl.ds(0, 1)], sb0)
        pltpu.async_copy(src_hbm.at[wid2, pl.ds(1, 1)], sb1, isem)
        pltpu.async_copy(x_hbm.at[sb0.at[0]], rows0, gsem)
        plsc.subcore_barrier()

        def step(c, buf, obuf, sb, osb, first, last, pf=True):
            if not first:
                pltpu.make_async_copy(obuf, acc.at[didx.at[c - 1]],
                                      ssem).wait()
            pltpu.make_async_copy(x_hbm.at[sb.at[0]], buf, gsem).wait()
            if not last:
                pltpu.make_async_copy(src_hbm.at[wid2, pl.ds(c + 1, 1)], osb,
                                      isem).wait()
                if pf:
                    pltpu.async_copy(src_hbm.at[wid2, pl.ds(c + 2, 1)], sb,
                                     isem)
                pltpu.async_copy(x_hbm.at[osb.at[0]], obuf, gsem)
            pltpu.async_copy(buf, acc.at[didx.at[c]], ssem, add=True)

        step(0, rows0, rows1, sb0, sb1, first=True, last=False)

        def pair(t, carry):
            c = 1 + 2 * t
            step(c, rows1, rows0, sb1, sb0, first=False, last=False)
            step(c + 1, rows0, rows1, sb0, sb1, first=False, last=False)
            return carry

        lax.fori_loop(0, (nck - 4) // 2, pair, 0)
        step(nck - 3, rows1, rows0, sb1, sb0, first=False, last=False)
        step(nck - 2, rows0, rows1, sb0, sb1, first=False, last=False,
             pf=False)
        step(nck - 1, rows1, rows0, sb1, sb0, first=False, last=True)
        pltpu.make_async_copy(rows1, acc.at[didx.at[nck - 1]],
                              ssem).wait()

        plsc.subcore_barrier()
        for b in range(_ZR // _K):
            r0 = sid * _ZR + b * _K
            pltpu.sync_copy(acc.at[pl.ds(r0, _K)], out_p.at[cid, pl.ds(r0, _K)])
        r0 = sid * _ZR + (_ZR // _K) * _K
        pltpu.sync_copy(acc.at[pl.ds(r0, rem)], out_p.at[cid, pl.ds(r0, rem)])

    return pl.kernel(body, out_type=out_type, mesh=mesh,
                     scratch_types=scratch)


_sc_agg = _make_sc_agg()


def _make_tc_hist():
    def body(d_ref, o_ref):
        d = d_ref[...]
        q = d // _D
        l = d % _D
        iq = lax.broadcasted_iota(jnp.int32, (_BE, _QH), 1)
        il = lax.broadcasted_iota(jnp.int32, (_BE, _D), 1)
        oq = (iq == q).astype(jnp.float32)
        ol = (il == l).astype(jnp.float32)
        part = lax.dot_general(oq, ol, (((0,), (0,)), ((), ())),
                               preferred_element_type=jnp.float32)

        @pl.when(pl.program_id(0) == 0)
        def _():
            o_ref[...] = jnp.zeros_like(o_ref)

        o_ref[...] += part

    return pl.pallas_call(
        body,
        grid=(_E // _BE,),
        in_specs=[pl.BlockSpec((_BE, 1), lambda i: (i, 0))],
        out_specs=pl.BlockSpec((_QH, _D), lambda i: (0, 0)),
        out_shape=jax.ShapeDtypeStruct((_QH, _D), jnp.float32),
    )


_DB = 8 * _D


def _make_tc_deg():
    def body(d_ref, o_ref):
        s = d_ref[...]
        r = lax.broadcasted_iota(jnp.int32, (_DB, 8), 0)
        q = lax.broadcasted_iota(jnp.int32, (_DB, 8), 1)
        sel = (q == r // _D).astype(jnp.float32)
        drow = lax.dot_general(sel, s, (((1,), (0,)), ((), ())),
                               preferred_element_type=jnp.float32)
        jj = lax.broadcasted_iota(jnp.int32, (_DB, _D), 0)
        ll = lax.broadcasted_iota(jnp.int32, (_DB, _D), 1)
        dsel = jnp.where(ll == jj % _D, drow, 0.0)
        o_ref[...] = jnp.sum(dsel, axis=1, keepdims=True)

    return pl.pallas_call(
        body,
        grid=(_QH // 8,),
        in_specs=[pl.BlockSpec((8, _D), lambda j: (j, 0))],
        out_specs=pl.BlockSpec((_DB, 1), lambda j: (j, 0)),
        out_shape=jax.ShapeDtypeStruct((_QH * _D, 1), jnp.float32),
    )


def _make_tc_layer(relu: bool):
    def body(p_ref, d_ref, x_ref, wl_ref, bl_ref, wr_ref, o_ref):
        agg = p_ref[0] + p_ref[1]
        mean = agg / jnp.maximum(d_ref[...], 1.0)
        h = lax.dot_general(mean, wl_ref[...], (((1,), (1,)), ((), ())),
                            preferred_element_type=jnp.float32)
        h = h + bl_ref[...]
        h = h + lax.dot_general(x_ref[...], wr_ref[...],
                                (((1,), (1,)), ((), ())),
                                preferred_element_type=jnp.float32)
        if relu:
            h = jnp.maximum(h, 0.0)
        o_ref[...] = h

    return pl.pallas_call(
        body,
        grid=(_N // _BM,),
        in_specs=[
            pl.BlockSpec((2, _BM, _D), lambda i: (0, i, 0)),
            pl.BlockSpec((_BM, 1), lambda i: (i, 0)),
            pl.BlockSpec((_BM, _D), lambda i: (i, 0)),
            pl.BlockSpec((_D, _D), lambda i: (0, 0)),
            pl.BlockSpec((1, _D), lambda i: (0, 0)),
            pl.BlockSpec((_D, _D), lambda i: (0, 0)),
        ],
        out_specs=pl.BlockSpec((_BM, _D), lambda i: (i, 0)),
        out_shape=jax.ShapeDtypeStruct((_N, _D), jnp.float32),
    )


_tc_hist = _make_tc_hist()
_tc_deg = _make_tc_deg()
_tc_relu = _make_tc_layer(relu=True)
_tc_lin = _make_tc_layer(relu=False)


def kernel(x, edge_index, W_l1, b_l1, W_r1, W_l2, b_l2, W_r2):
    src = edge_index[0]
    dst = edge_index[1]
    pad = _E_PAD - _E
    src_r = jnp.concatenate([src, jnp.zeros((pad,), jnp.int32)])
    dst_r = jnp.concatenate([dst, jnp.full((pad,), _N, jnp.int32)])
    def _to_tiles(a):
        flat = a.reshape(_NS * (_C0 + _C1), _K)
        t0 = flat[:_NS * _C0].reshape(_NS, _C0, _K)
        t0 = jnp.pad(t0, ((0, 0), (0, _CMX - _C0), (0, 0)))
        t1 = flat[_NS * _C0:].reshape(_NS, _C1, _K)
        t1 = jnp.pad(t1, ((0, 0), (0, _CMX - _C1), (0, 0)))
        return jnp.concatenate([t0, t1])
    src_r = _to_tiles(src_r)
    dst_r = _to_tiles(dst_r)

    dcol = _tc_deg(_tc_hist(dst.reshape(_E, 1)))
    p1 = _sc_agg(x, src_r, dst_r)
    h = _tc_relu(p1, dcol, x, W_l1, b_l1.reshape(1, _D), W_r1)
    p2 = _sc_agg(h, src_r, dst_r)
    out = _tc_lin(p2, dcol, h, W_l2, b_l2.reshape(1, _D), W_r2)
    return out


# speedup vs baseline: 1.0401x; 1.0401x over previous
import jax
import jax.numpy as jnp
from jax import lax
from jax.experimental import pallas as pl
from jax.experimental.pallas import tpu as pltpu
from jax.experimental.pallas import tpu_sc as plsc

_N = 10000
_E = 320000
_D = 128

_NC = 2
_NS = 16
_NW = _NC * _NS
_K = 112
_EPT = -(-_E // (_NW * _K)) * _K
_E_PAD = _EPT * _NW
_CHUNKS = _EPT // _K
_C0 = 120
_C1 = 60
_CMX = max(_C0, _C1)
_NP = 10112
_ZR = _NP // _NS
_BM = 1000
_QH = 80
_BE = 4000


def _make_sc_agg():
    mesh = plsc.VectorSubcoreMesh(core_axis_name="c", subcore_axis_name="s")
    out_type = jax.ShapeDtypeStruct((_NC, _NP, _D), jnp.float32)
    scratch = [
        pltpu.VMEM((1, _K), jnp.int32),
        pltpu.VMEM((1, _K), jnp.int32),
        pltpu.VMEM((_CMX, _K), jnp.int32),
        pltpu.VMEM((_K, _D), jnp.float32),
        pltpu.VMEM((_K, _D), jnp.float32),
        pltpu.VMEM_SHARED((_NP, _D), jnp.float32),
        pltpu.SemaphoreType.DMA,
        pltpu.SemaphoreType.DMA,
        pltpu.SemaphoreType.DMA,
    ]

    def body(x_hbm, src_hbm, dst_hbm, out_p, sb0, sb1, didx, rows0, rows1,
             acc, gsem, ssem, isem):
        cid = lax.axis_index("c")
        sid = lax.axis_index("s")
        wid = sid * _NC + cid

        z16 = jnp.zeros((16,), jnp.float32)

        def zrow(i, carry):
            for j in range(_D // 16):
                rows0[i, pl.ds(j * 16, 16)] = z16
            return carry

        lax.fori_loop(0, _K, zrow, 0)
        for b in range(_ZR // _K):
            pltpu.sync_copy(rows0, acc.at[pl.ds(sid * _ZR + b * _K, _K)])
        rem = _ZR % _K
        pltpu.sync_copy(rows0.at[pl.ds(0, rem)],
                        acc.at[pl.ds(sid * _ZR + (_ZR // _K) * _K, rem)])
        wid2 = cid * _NS + sid
        nck = jnp.where(cid == 0, _C0, _C1)
        pltpu.sync_copy(dst_hbm.at[wid2], didx)
        pltpu.sync_copy(src_hbm.at[wid2, pl.ds(0, 1)], sb0)
        pltpu.async_copy(src_hbm.at[wid2, pl.ds(1, 1)], sb1, isem)
        pltpu.async_copy(x_hbm.at[sb0.at[0]], rows0, gsem)
        plsc.subcore_barrier()

        def step(c, buf, obuf, sb, osb, first, last, pf=True):
            if not first:
                pltpu.make_async_copy(obuf, acc.at[didx.at[c - 1]],
                                      ssem).wait()
            pltpu.make_async_copy(x_hbm.at[sb.at[0]], buf, gsem).wait()
            if not last:
                pltpu.make_async_copy(src_hbm.at[wid2, pl.ds(c + 1, 1)], osb,
                                      isem).wait()
                if pf:
                    pltpu.async_copy(src_hbm.at[wid2, pl.ds(c + 2, 1)], sb,
                                     isem)
                pltpu.async_copy(x_hbm.at[osb.at[0]], obuf, gsem)
            pltpu.async_copy(buf, acc.at[didx.at[c]], ssem, add=True)

        step(0, rows0, rows1, sb0, sb1, first=True, last=False)

        def pair(t, carry):
            c = 1 + 2 * t
            step(c, rows1, rows0, sb1, sb0, first=False, last=False)
            step(c + 1, rows0, rows1, sb0, sb1, first=False, last=False)
            return carry

        lax.fori_loop(0, (nck - 4) // 2, pair, 0)
        step(nck - 3, rows1, rows0, sb1, sb0, first=False, last=False)
        step(nck - 2, rows0, rows1, sb0, sb1, first=False, last=False,
             pf=False)
        step(nck - 1, rows1, rows0, sb1, sb0, first=False, last=True)
        pltpu.make_async_copy(rows1, acc.at[didx.at[nck - 1]],
                              ssem).wait()

        plsc.subcore_barrier()
        for b in range(_ZR // _K):
            r0 = sid * _ZR + b * _K
            pltpu.sync_copy(acc.at[pl.ds(r0, _K)], out_p.at[cid, pl.ds(r0, _K)])
        r0 = sid * _ZR + (_ZR // _K) * _K
        pltpu.sync_copy(acc.at[pl.ds(r0, rem)], out_p.at[cid, pl.ds(r0, rem)])

    return pl.kernel(body, out_type=out_type, mesh=mesh,
                     scratch_types=scratch)


_sc_agg = _make_sc_agg()


def _make_tc_hist():
    def body(d_ref, o_ref):
        d = d_ref[...]
        q = d // _D
        l = d % _D
        iq = lax.broadcasted_iota(jnp.int32, (_BE, _QH), 1)
        il = lax.broadcasted_iota(jnp.int32, (_BE, _D), 1)
        oq = (iq == q).astype(jnp.float32)
        ol = (il == l).astype(jnp.float32)
        part = lax.dot_general(oq, ol, (((0,), (0,)), ((), ())),
                               preferred_element_type=jnp.float32)

        @pl.when(pl.program_id(0) == 0)
        def _():
            o_ref[...] = jnp.zeros_like(o_ref)

        o_ref[...] += part

    return pl.pallas_call(
        body,
        grid=(_E // _BE,),
        in_specs=[pl.BlockSpec((_BE, 1), lambda i: (i, 0))],
        out_specs=pl.BlockSpec((_QH, _D), lambda i: (0, 0)),
        out_shape=jax.ShapeDtypeStruct((_QH, _D), jnp.float32),
    )


_DB = 8 * _D


def _make_tc_deg():
    def body(d_ref, o_ref):
        s = d_ref[...]
        r = lax.broadcasted_iota(jnp.int32, (_DB, 8), 0)
        q = lax.broadcasted_iota(jnp.int32, (_DB, 8), 1)
        sel = (q == r // _D).astype(jnp.float32)
        drow = lax.dot_general(sel, s, (((1,), (0,)), ((), ())),
                               preferred_element_type=jnp.float32)
        jj = lax.broadcasted_iota(jnp.int32, (_DB, _D), 0)
        ll = lax.broadcasted_iota(jnp.int32, (_DB, _D), 1)
        dsel = jnp.where(ll == jj % _D, drow, 0.0)
        o_ref[...] = jnp.sum(dsel, axis=1, keepdims=True)

    return pl.pallas_call(
        body,
        grid=(_QH // 8,),
        in_specs=[pl.BlockSpec((8, _D), lambda j: (j, 0))],
        out_specs=pl.BlockSpec((_DB, 1), lambda j: (j, 0)),
        out_shape=jax.ShapeDtypeStruct((_QH * _D, 1), jnp.float32),
    )


def _make_tc_layer(relu: bool):
    def body(p_ref, d_ref, x_ref, wl_ref, bl_ref, wr_ref, o_ref):
        agg = p_ref[0] + p_ref[1]
        mean = agg / jnp.maximum(d_ref[...], 1.0)
        h = lax.dot_general(mean, wl_ref[...], (((1,), (1,)), ((), ())),
                            preferred_element_type=jnp.float32)
        h = h + bl_ref[...]
        h = h + lax.dot_general(x_ref[...], wr_ref[...],
                                (((1,), (1,)), ((), ())),
                                preferred_element_type=jnp.float32)
        if relu:
            h = jnp.maximum(h, 0.0)
        o_ref[...] = h

    return pl.pallas_call(
        body,
        grid=(_N // _BM,),
        in_specs=[
            pl.BlockSpec((2, _BM, _D), lambda i: (0, i, 0)),
            pl.BlockSpec((_BM, 1), lambda i: (i, 0)),
            pl.BlockSpec((_BM, _D), lambda i: (i, 0)),
            pl.BlockSpec((_D, _D), lambda i: (0, 0)),
            pl.BlockSpec((1, _D), lambda i: (0, 0)),
            pl.BlockSpec((_D, _D), lambda i: (0, 0)),
        ],
        out_specs=pl.BlockSpec((_BM, _D), lambda i: (i, 0)),
        out_shape=jax.ShapeDtypeStruct((_N, _D), jnp.float32),
    )


_tc_hist = _make_tc_hist()
_tc_deg = _make_tc_deg()
_tc_relu = _make_tc_layer(relu=True)
_tc_lin = _make_tc_layer(relu=False)


def kernel(x, edge_index, W_l1, b_l1, W_r1, W_l2, b_l2, W_r2):
    src = edge_index[0]
    dst = edge_index[1]
    pad = _E_PAD - _E
    src_r = jnp.concatenate([src, jnp.zeros((pad,), jnp.int32)])
    dst_r = jnp.concatenate([dst, jnp.full((pad,), _N, jnp.int32)])
    def _to_tiles(a):
        flat = a.reshape(_NS * (_C0 + _C1), _K)
        t0 = flat[:_NS * _C0].reshape(_NS, _C0, _K)
        t0 = jnp.pad(t0, ((0, 0), (0, _CMX - _C0), (0, 0)))
        t1 = flat[_NS * _C0:].reshape(_NS, _C1, _K)
        t1 = jnp.pad(t1, ((0, 0), (0, _CMX - _C1), (0, 0)))
        return jnp.concatenate([t0, t1])
    src_r = _to_tiles(src_r)
    dst_r = _to_tiles(dst_r)

    dcol = _tc_deg(_tc_hist(dst.reshape(_E, 1)))
    p1 = _sc_agg(x, src_r, dst_r)
    h = _tc_relu(p1, dcol, x, W_l1, b_l1.reshape(1, _D), W_r1)
    p2 = _sc_agg(h, src_r, dst_r)
    out = _tc_lin(p2, dcol, h, W_l2, b_l2.reshape(1, _D), W_r2)
    return out


# speedup vs baseline: 1.0642x; 1.0231x over previous
import jax
import jax.numpy as jnp
from jax import lax
from jax.experimental import pallas as pl
from jax.experimental.pallas import tpu as pltpu
from jax.experimental.pallas import tpu_sc as plsc

_N = 10000
_E = 320000
_D = 128

_NC = 2
_NS = 16
_NW = _NC * _NS
_K = 112
_EPT = -(-_E // (_NW * _K)) * _K
_E_PAD = _EPT * _NW
_CHUNKS = _EPT // _K
_C0 = 114
_C1 = 66
_CMX = max(_C0, _C1)
_NP = 10112
_ZR = _NP // _NS
_BM = 1000
_QH = 80
_BE = 4000


def _make_sc_agg():
    mesh = plsc.VectorSubcoreMesh(core_axis_name="c", subcore_axis_name="s")
    out_type = jax.ShapeDtypeStruct((_NC, _NP, _D), jnp.float32)
    scratch = [
        pltpu.VMEM((1, _K), jnp.int32),
        pltpu.VMEM((1, _K), jnp.int32),
        pltpu.VMEM((_CMX, _K), jnp.int32),
        pltpu.VMEM((_K, _D), jnp.float32),
        pltpu.VMEM((_K, _D), jnp.float32),
        pltpu.VMEM_SHARED((_NP, _D), jnp.float32),
        pltpu.SemaphoreType.DMA,
        pltpu.SemaphoreType.DMA,
        pltpu.SemaphoreType.DMA,
    ]

    def body(x_hbm, src_hbm, dst_hbm, out_p, sb0, sb1, didx, rows0, rows1,
             acc, gsem, ssem, isem):
        cid = lax.axis_index("c")
        sid = lax.axis_index("s")
        wid = sid * _NC + cid

        z16 = jnp.zeros((16,), jnp.float32)

        def zrow(i, carry):
            for j in range(_D // 16):
                rows0[i, pl.ds(j * 16, 16)] = z16
            return carry

        lax.fori_loop(0, _K, zrow, 0)
        for b in range(_ZR // _K):
            pltpu.sync_copy(rows0, acc.at[pl.ds(sid * _ZR + b * _K, _K)])
        rem = _ZR % _K
        pltpu.sync_copy(rows0.at[pl.ds(0, rem)],
                        acc.at[pl.ds(sid * _ZR + (_ZR // _K) * _K, rem)])
        wid2 = cid * _NS + sid
        nck = jnp.where(cid == 0, _C0, _C1)
        pltpu.sync_copy(dst_hbm.at[wid2], didx)
        pltpu.sync_copy(src_hbm.at[wid2, pl.ds(0, 1)], sb0)
        pltpu.async_copy(src_hbm.at[wid2, pl.ds(1, 1)], sb1, isem)
        pltpu.async_copy(x_hbm.at[sb0.at[0]], rows0, gsem)
        plsc.subcore_barrier()

        def step(c, buf, obuf, sb, osb, first, last, pf=True):
            if not first:
                pltpu.make_async_copy(obuf, acc.at[didx.at[c - 1]],
                                      ssem).wait()
            pltpu.make_async_copy(x_hbm.at[sb.at[0]], buf, gsem).wait()
            if not last:
                pltpu.make_async_copy(src_hbm.at[wid2, pl.ds(c + 1, 1)], osb,
                                      isem).wait()
                if pf:
                    pltpu.async_copy(src_hbm.at[wid2, pl.ds(c + 2, 1)], sb,
                                     isem)
                pltpu.async_copy(x_hbm.at[osb.at[0]], obuf, gsem)
            pltpu.async_copy(buf, acc.at[didx.at[c]], ssem, add=True)

        step(0, rows0, rows1, sb0, sb1, first=True, last=False)

        def pair(t, carry):
            c = 1 + 2 * t
            step(c, rows1, rows0, sb1, sb0, first=False, last=False)
            step(c + 1, rows0, rows1, sb0, sb1, first=False, last=False)
            return carry

        lax.fori_loop(0, (nck - 4) // 2, pair, 0)
        step(nck - 3, rows1, rows0, sb1, sb0, first=False, last=False)
        step(nck - 2, rows0, rows1, sb0, sb1, first=False, last=False,
             pf=False)
        step(nck - 1, rows1, rows0, sb1, sb0, first=False, last=True)
        pltpu.make_async_copy(rows1, acc.at[didx.at[nck - 1]],
                              ssem).wait()

        plsc.subcore_barrier()
        for b in range(_ZR // _K):
            r0 = sid * _ZR + b * _K
            pltpu.sync_copy(acc.at[pl.ds(r0, _K)], out_p.at[cid, pl.ds(r0, _K)])
        r0 = sid * _ZR + (_ZR // _K) * _K
        pltpu.sync_copy(acc.at[pl.ds(r0, rem)], out_p.at[cid, pl.ds(r0, rem)])

    return pl.kernel(body, out_type=out_type, mesh=mesh,
                     scratch_types=scratch)


_sc_agg = _make_sc_agg()


_DB = 8 * _D


def _make_tc_deg():
    def body(d_ref, o_ref):
        s = d_ref[...]
        r = lax.broadcasted_iota(jnp.int32, (_DB, 8), 0)
        q = lax.broadcasted_iota(jnp.int32, (_DB, 8), 1)
        sel = (q == r // _D).astype(jnp.float32)
        drow = lax.dot_general(sel, s, (((1,), (0,)), ((), ())),
                               preferred_element_type=jnp.float32)
        jj = lax.broadcasted_iota(jnp.int32, (_DB, _D), 0)
        ll = lax.broadcasted_iota(jnp.int32, (_DB, _D), 1)
        dsel = jnp.where(ll == jj % _D, drow, 0.0)
        o_ref[...] = jnp.sum(dsel, axis=1, keepdims=True)

    return pl.pallas_call(
        body,
        grid=(_QH // 8,),
        in_specs=[pl.BlockSpec((8, _D), lambda j: (j, 0))],
        out_specs=pl.BlockSpec((_DB, 1), lambda j: (j, 0)),
        out_shape=jax.ShapeDtypeStruct((_QH * _D, 1), jnp.float32),
    )


def _make_tc_layer(relu: bool):
    def body(p_ref, d_ref, x_ref, wl_ref, bl_ref, wr_ref, o_ref):
        agg = p_ref[0] + p_ref[1]
        mean = agg / jnp.maximum(d_ref[...], 1.0)
        h = lax.dot_general(mean, wl_ref[...], (((1,), (1,)), ((), ())),
                            preferred_element_type=jnp.float32)
        h = h + bl_ref[...]
        h = h + lax.dot_general(x_ref[...], wr_ref[...],
                                (((1,), (1,)), ((), ())),
                                preferred_element_type=jnp.float32)
        if relu:
            h = jnp.maximum(h, 0.0)
        o_ref[...] = h

    return pl.pallas_call(
        body,
        grid=(_N // _BM,),
        in_specs=[
            pl.BlockSpec((2, _BM, _D), lambda i: (0, i, 0)),
            pl.BlockSpec((_BM, 1), lambda i: (i, 0)),
            pl.BlockSpec((_BM, _D), lambda i: (i, 0)),
            pl.BlockSpec((_D, _D), lambda i: (0, 0)),
            pl.BlockSpec((1, _D), lambda i: (0, 0)),
            pl.BlockSpec((_D, _D), lambda i: (0, 0)),
        ],
        out_specs=pl.BlockSpec((_BM, _D), lambda i: (i, 0)),
        out_shape=jax.ShapeDtypeStruct((_N, _D), jnp.float32),
    )


def _make_tc_hist():
    def body(d_ref, o_ref):
        d = d_ref[...]
        q = d // _D
        l = d % _D
        iq = lax.broadcasted_iota(jnp.int32, (_BE, _QH), 1)
        il = lax.broadcasted_iota(jnp.int32, (_BE, _D), 1)
        oq = (iq == q).astype(jnp.float32)
        ol = (il == l).astype(jnp.float32)
        part = lax.dot_general(oq, ol, (((0,), (0,)), ((), ())),
                               preferred_element_type=jnp.float32)

        @pl.when(pl.program_id(0) == 0)
        def _():
            o_ref[...] = jnp.zeros_like(o_ref)

        o_ref[...] += part

    return pl.pallas_call(
        body,
        grid=(_E // _BE,),
        in_specs=[pl.BlockSpec((_BE, 1), lambda i: (i, 0))],
        out_specs=pl.BlockSpec((_QH, _D), lambda i: (0, 0)),
        out_shape=jax.ShapeDtypeStruct((_QH, _D), jnp.float32),
    )


_tc_hist = _make_tc_hist()
_tc_deg = _make_tc_deg()
_tc_relu = _make_tc_layer(relu=True)
_tc_lin = _make_tc_layer(relu=False)


def kernel(x, edge_index, W_l1, b_l1, W_r1, W_l2, b_l2, W_r2):
    src = edge_index[0]
    dst = edge_index[1]
    pad = _E_PAD - _E
    src_r = jnp.concatenate([src, jnp.zeros((pad,), jnp.int32)])
    dst_r = jnp.concatenate([dst, jnp.full((pad,), _N, jnp.int32)])
    def _to_tiles(a):
        flat = a.reshape(_NS * (_C0 + _C1), _K)
        t0 = flat[:_NS * _C0].reshape(_NS, _C0, _K)
        t0 = jnp.pad(t0, ((0, 0), (0, _CMX - _C0), (0, 0)))
        t1 = flat[_NS * _C0:].reshape(_NS, _C1, _K)
        t1 = jnp.pad(t1, ((0, 0), (0, _CMX - _C1), (0, 0)))
        return jnp.concatenate([t0, t1])
    src_r = _to_tiles(src_r)
    dst_r = _to_tiles(dst_r)

    dcol = _tc_deg(_tc_hist(dst.reshape(_E, 1)))
    p1 = _sc_agg(x, src_r, dst_r)
    h = _tc_relu(p1, dcol, x, W_l1, b_l1.reshape(1, _D), W_r1)
    p2 = _sc_agg(h, src_r, dst_r)
    out = _tc_lin(p2, dcol, h, W_l2, b_l2.reshape(1, _D), W_r2)
    return out
